# baseline (speedup 1.0000x reference)
import functools

import jax
import jax.numpy as jnp
from jax import lax
from jax.experimental import pallas as pl
from jax.experimental.pallas import tpu as pltpu
from jax.experimental.pallas import tpu_sc as plsc

_N = 256
_K = 128
_D = 96
_NW = 32
_SLAB = _K * _N // 4


def _sortable_key(sim):
    i = lax.bitcast_convert_type(sim, jnp.int32)
    return jnp.where(i >= 0, i, i ^ jnp.int32(0x7FFFFFFF))


def _bitonic_argsort_cols(sim):
    n = sim.shape[0]
    key = _sortable_key(sim)
    idx = jax.lax.broadcasted_iota(jnp.int32, sim.shape, 0)
    pos = jax.lax.broadcasted_iota(jnp.int32, (n, 1), 0)

    size = 2
    while size <= n:
        d = size // 2
        dir_up = (pos & size) == 0
        while d >= 1:
            upper = (pos & d) != 0
            k_m = jnp.roll(key, -d, axis=0)
            k_p = jnp.roll(key, d, axis=0)
            i_m = jnp.roll(idx, -d, axis=0)
            i_p = jnp.roll(idx, d, axis=0)
            k_part = jnp.where(upper, k_p, k_m)
            i_part = jnp.where(upper, i_p, i_m)
            before = (key > k_part) | ((key == k_part) & (idx < i_part))
            keep = before ^ upper ^ (~dir_up)
            key = jnp.where(keep, key, k_part)
            idx = jnp.where(keep, idx, i_part)
            d //= 2
        size *= 2
    return idx


def _sort_body(x_ref, idx_ref):
    x = x_ref[0]
    dots = lax.dot_general(x, x, (((1,), (1,)), ((), ())),
                           precision=lax.Precision.DEFAULT,
                           preferred_element_type=jnp.float32)
    n2 = jnp.sum(x * x, axis=1, keepdims=True)
    norms = jnp.sqrt(n2)
    outer = norms * jnp.transpose(norms)
    sim = dots / jnp.maximum(outer, jnp.float32(1e-8))

    strip = 128
    for s0 in range(0, _N, strip):
        idx_sorted = _bitonic_argsort_cols(sim[:, s0:s0 + strip])
        idx_ref[0, :, s0:s0 + strip] = idx_sorted[:_K, :]


def _counts_sc_kernel(idx_hbm, c_hbm, idx_v, c_v):
    wid = lax.axis_index("s") * 2 + lax.axis_index("c")
    base = wid * _SLAB
    pltpu.sync_copy(idx_hbm.at[pl.ds(base, _SLAB)], idx_v)

    iota = lax.iota(jnp.int32, 16)
    zeros = jnp.zeros((16,), jnp.float32)
    ones = jnp.ones((16,), jnp.float32)

    def zero_body(m, _):
        c_v[pl.ds(m * 16, 16)] = zeros
        return 0

    lax.fori_loop(0, _SLAB // 16, zero_body, 0)

    def body(i, _):
        for t0 in (0, 16):
            tvec = (t0 + iota) * _N
            cols = plsc.load_gather(idx_v, [tvec + i])
            plsc.addupdate_scatter(c_v, [tvec + cols], ones)
        return 0

    lax.fori_loop(0, _N, body, 0)
    pltpu.sync_copy(c_v, c_hbm.at[pl.ds(base, _SLAB)])


@functools.partial(
    pl.kernel,
    mesh=plsc.VectorSubcoreMesh(core_axis_name="c", subcore_axis_name="s"),
    out_type=jax.ShapeDtypeStruct((8 * _K * _N,), jnp.float32),
    compiler_params=pltpu.CompilerParams(needs_layout_passes=False),
    scratch_types=[
        pltpu.VMEM((_SLAB,), jnp.int32),
        pltpu.VMEM((_SLAB,), jnp.float32),
    ],
)
def _counts_sc(idx_hbm, c_hbm, idx_v, c_v):
    _counts_sc_kernel(idx_hbm, c_hbm, idx_v, c_v)


def _merge_body(c_ref, x_ref, out_ref):
    out = lax.dot_general(c_ref[0], x_ref[0], (((1,), (0,)), ((), ())),
                          preferred_element_type=jnp.float32)
    out_ref[0, 0] = out * jnp.float32(1.0 / _N)


@jax.jit
def kernel(clip_embeddings):
    b, n, d = clip_embeddings.shape
    idx_top = pl.pallas_call(
        _sort_body,
        grid=(b,),
        in_specs=[pl.BlockSpec((1, n, d), lambda i: (i, 0, 0))],
        out_specs=pl.BlockSpec((1, _K, n), lambda i: (i, 0, 0)),
        out_shape=jax.ShapeDtypeStruct((b, _K, n), jnp.int32),
        compiler_params=pltpu.CompilerParams(
            dimension_semantics=("arbitrary",),
        ),
    )(clip_embeddings)
    c = _counts_sc(idx_top.reshape(b * _K * n)).reshape(b, _K, n)
    return pl.pallas_call(
        _merge_body,
        grid=(b,),
        in_specs=[pl.BlockSpec((1, _K, n), lambda i: (i, 0, 0)),
                  pl.BlockSpec((1, n, d), lambda i: (i, 0, 0))],
        out_specs=pl.BlockSpec((1, 1, _K, d), lambda i: (i, 0, 0, 0)),
        out_shape=jax.ShapeDtypeStruct((b, 1, _K, d), jnp.float32),
        compiler_params=pltpu.CompilerParams(
            dimension_semantics=("arbitrary",),
        ),
    )(c, clip_embeddings)


# speedup vs baseline: 1.0328x; 1.0328x over previous
import functools

import jax
import jax.numpy as jnp
from jax import lax
from jax.experimental import pallas as pl
from jax.experimental.pallas import tpu as pltpu
from jax.experimental.pallas import tpu_sc as plsc

_N = 256
_K = 128
_D = 96
_NW = 32
_SLAB = _K * _N // 4


def _sortable_key(sim):
    i = lax.bitcast_convert_type(sim, jnp.int32)
    return jnp.where(i >= 0, i, i ^ jnp.int32(0x7FFFFFFF))


def _bitonic_argsort_cols(sim):
    n = sim.shape[0]
    key = _sortable_key(sim)
    idx = jax.lax.broadcasted_iota(jnp.int32, sim.shape, 0)
    pos = jax.lax.broadcasted_iota(jnp.int32, (n, 1), 0)

    size = 2
    while size <= n:
        d = size // 2
        dir_up = (pos & size) == 0
        while d >= 1:
            upper = (pos & d) != 0
            k_m = jnp.roll(key, -d, axis=0)
            k_p = jnp.roll(key, d, axis=0)
            i_m = jnp.roll(idx, -d, axis=0)
            i_p = jnp.roll(idx, d, axis=0)
            k_part = jnp.where(upper, k_p, k_m)
            i_part = jnp.where(upper, i_p, i_m)
            before = (key > k_part) | ((key == k_part) & (idx < i_part))
            keep = before ^ upper ^ (~dir_up)
            key = jnp.where(keep, key, k_part)
            idx = jnp.where(keep, idx, i_part)
            d //= 2
        size *= 2
    return idx


def _sort_body(x_ref, idx_ref):
    x = x_ref[0]
    dots = lax.dot_general(x, x, (((1,), (1,)), ((), ())),
                           precision=lax.Precision.DEFAULT,
                           preferred_element_type=jnp.float32)
    n2 = jnp.sum(x * x, axis=1, keepdims=True)
    norms = jnp.sqrt(n2)
    outer = norms * jnp.transpose(norms)
    sim = dots / jnp.maximum(outer, jnp.float32(1e-8))

    strip = 128
    for s0 in range(0, _N, strip):
        idx_sorted = _bitonic_argsort_cols(sim[:, s0:s0 + strip])
        idx_ref[0, :, s0:s0 + strip] = idx_sorted[:_K, :]


_HSLAB = 4 * _K * _N // _NW


@functools.partial(
    pl.kernel,
    mesh=plsc.VectorSubcoreMesh(core_axis_name="c", subcore_axis_name="s"),
    out_type=jax.ShapeDtypeStruct((4 * _K * _N,), jnp.float32),
    compiler_params=pltpu.CompilerParams(needs_layout_passes=False),
    scratch_types=[
        pltpu.VMEM((_HSLAB,), jnp.int32),
        pltpu.VMEM((_HSLAB,), jnp.float32),
    ],
)
def _counts_sc(idx_hbm, c_hbm, idx_v, c_v):
    wid = lax.axis_index("s") * 2 + lax.axis_index("c")
    base = wid * _HSLAB
    pltpu.sync_copy(idx_hbm.at[pl.ds(base, _HSLAB)], idx_v)

    iota = lax.iota(jnp.int32, 16)
    zeros = jnp.zeros((16,), jnp.float32)
    ones = jnp.ones((16,), jnp.float32)
    tvec = iota * _N

    def zero_body(m, _):
        for u in range(4):
            c_v[pl.ds((m * 4 + u) * 16, 16)] = zeros
        return 0

    lax.fori_loop(0, _HSLAB // 64, zero_body, 0)

    def body(i, _):
        for u in range(4):
            cols = plsc.load_gather(idx_v, [tvec + (i * 4 + u)])
            plsc.addupdate_scatter(c_v, [tvec + cols], ones)
        return 0

    lax.fori_loop(0, _N // 4, body, 0)
    pltpu.sync_copy(c_v, c_hbm.at[pl.ds(base, _HSLAB)])


def _merge_body(c_ref, x_ref, out_ref):
    out = lax.dot_general(c_ref[0], x_ref[0], (((1,), (0,)), ((), ())),
                          preferred_element_type=jnp.float32)
    out_ref[0, 0] = out * jnp.float32(1.0 / _N)


def _sort_half(x_half):
    hb, n, d = x_half.shape
    return pl.pallas_call(
        _sort_body,
        grid=(hb,),
        in_specs=[pl.BlockSpec((1, n, d), lambda i: (i, 0, 0))],
        out_specs=pl.BlockSpec((1, _K, n), lambda i: (i, 0, 0)),
        out_shape=jax.ShapeDtypeStruct((hb, _K, n), jnp.int32),
        compiler_params=pltpu.CompilerParams(
            dimension_semantics=("arbitrary",),
        ),
    )(x_half)


@jax.jit
def kernel(clip_embeddings):
    b, n, d = clip_embeddings.shape
    hb = b // 2
    idx_a = _sort_half(clip_embeddings[:hb])
    c_a = _counts_sc(idx_a.reshape(hb * _K * n))
    idx_b = _sort_half(clip_embeddings[hb:])
    c_b = _counts_sc(idx_b.reshape(hb * _K * n))
    c = jnp.concatenate(
        [c_a.reshape(hb, _K, n), c_b.reshape(hb, _K, n)], axis=0)
    return pl.pallas_call(
        _merge_body,
        grid=(b,),
        in_specs=[pl.BlockSpec((1, _K, n), lambda i: (i, 0, 0)),
                  pl.BlockSpec((1, n, d), lambda i: (i, 0, 0))],
        out_specs=pl.BlockSpec((1, 1, _K, d), lambda i: (i, 0, 0, 0)),
        out_shape=jax.ShapeDtypeStruct((b, 1, _K, d), jnp.float32),
        compiler_params=pltpu.CompilerParams(
            dimension_semantics=("arbitrary",),
        ),
    )(c, clip_embeddings)
